# baseline (speedup 1.0000x reference)
import dataclasses

import jax
import jax.numpy as jnp
from jax import lax
from jax.experimental import pallas as pl
from jax.experimental.pallas import tpu as pltpu
from jax.experimental.pallas import tpu_sc as plsc

N_USER_ = 50000
N_ITEM_ = 50000
N_ = N_USER_ + N_ITEM_
D_ = 32
D2_ = 16
K_ = 10
E_ = 1600000

NC_ = 2
NS_ = 16
HALF_ = N_ // NC_
ACC_R_ = 51200
TRASH_ = HALF_

EBLK_ = 128
SBB_ = 40
SBE_ = SBB_ * EBLK_
NSB_ = 20
EB_T_ = NSB_ * SBE_
PAD_E_ = NS_ * EB_T_

NBLK_ = HALF_ // 16
NJ_ = 196


def _body(embA_h, embB_h, src_h, dst_h, val_h, nbr_h, wts_h,
          outA_h, outB_h,
          acc,
          msrc0, msrc1, mdst0, mdst1, mval0, mval1,
          rows0, rows1, rows2, rows3,
          osb0, osb1, osb2, osb3,
          lidx0, lidx1, lidx2, lidx3,
          nidx0, nidx1, nidx2, nidx3, nwb0, nwb1, nwb2, nwb3,
          neb0, neb1, neb2, neb3, nab0, nab1, nab2, nab3,
          nr0, nr1, nr2, nr3, obuf0, obuf1,
          gs0, gs1, gs2, gs3, ss0, ss1, ss2, ss3,
          ms0, ms1, ngs0, ngs1, ngs2, ngs3,
          nms0, nms1, nms2, nms3, os0, os1):
    msrc = [msrc0, msrc1]
    mdst = [mdst0, mdst1]
    mval = [mval0, mval1]
    rows = [rows0, rows1, rows2, rows3]
    osb = [osb0, osb1, osb2, osb3]
    lidx = [lidx0, lidx1, lidx2, lidx3]
    nidx = [nidx0, nidx1, nidx2, nidx3]
    nwb = [nwb0, nwb1, nwb2, nwb3]
    neb = [neb0, neb1, neb2, neb3]
    nab = [nab0, nab1, nab2, nab3]
    nrows = [nr0, nr1, nr2, nr3]
    obuf = [obuf0, obuf1]
    gs = [gs0, gs1, gs2, gs3]
    ss = [ss0, ss1, ss2, ss3]
    ms = [ms0, ms1]
    ngs = [ngs0, ngs1, ngs2, ngs3]
    nms = [nms0, nms1, nms2, nms3]
    osm = [os0, os1]

    c = lax.axis_index("c")
    s = lax.axis_index("s")
    cbase = c * HALF_
    tile_e0 = s * EB_T_

    def _splat(i):
        return jnp.full((16,), i, jnp.int32)

    def _zero_acc():
        zero16 = jnp.zeros((16,), jnp.float32)

        @pl.loop(0, EBLK_)
        def _(r):
            osb0[r, pl.ds(0, 16)] = zero16

        @pl.loop(0, ACC_R_ // NS_ // EBLK_)
        def _(jz):
            row0 = pl.multiple_of(s * (ACC_R_ // NS_) + jz * EBLK_, EBLK_)
            pltpu.sync_copy(osb0, acc.at[pl.ds(row0, EBLK_)])

    def _e_load_meta(j, jj):
        mb = pl.multiple_of((tile_e0 + j * SBE_) // EBLK_, SBB_)
        pltpu.sync_copy(src_h.at[pl.ds(mb, SBB_)], msrc[jj])
        pltpu.sync_copy(dst_h.at[pl.ds(mb, SBB_)], mdst[jj])
        pltpu.sync_copy(val_h.at[pl.ds(mb, SBB_)], mval[jj])

    def _edge_pass(emb_h):
        def _superblock(jo, jj):
            j = jo * 2 + jj
            _e_load_meta(j, jj)

            @pl.loop(0, SBB_)
            def _(b):
                pltpu.sync_copy(emb_h.at[msrc[jj].at[b]], rows0)

                @pl.loop(0, 8)
                def _(g):
                    off = pl.multiple_of(g * 16, 16)
                    d = mdst[jj][b, pl.ds(off, 16)]
                    loc = d - cbase
                    ok = (loc >= 0) & (loc < HALF_)
                    lidx0[pl.ds(off, 16)] = jnp.where(ok, loc, TRASH_)
                    for e in range(16):
                        r = off + e
                        bc = plsc.load_gather(mval[jj],
                                              [_splat(b), _splat(r)])
                        osb0[r, pl.ds(0, 16)] = (
                            rows0[r, pl.ds(0, 16)] * bc)

                pltpu.sync_copy(osb0, acc.at[lidx0], add=True)

        @pl.loop(0, NSB_ // 2)
        def _(jo):
            _superblock(jo, 0)
            _superblock(jo, 1)

    def _node_pass(emb_h, out_h):
        def _load_meta(t, p):
            lbase = pl.multiple_of(t * 16, 16)
            ibase = pl.multiple_of((c * HALF_ + lbase) * K_, 32)
            gbase = pl.multiple_of(cbase + lbase, 16)
            pltpu.sync_copy(nbr_h.at[pl.ds(ibase, 16 * K_)], nidx[p])
            pltpu.sync_copy(wts_h.at[pl.ds(ibase, 16 * K_)], nwb[p])
            pltpu.sync_copy(emb_h.at[pl.ds(gbase, 16)], neb[p])
            pltpu.sync_copy(acc.at[pl.ds(lbase, 16)], nab[p])

        def _block(jo, pi):
            j = jo * 4 + pi
            t = s + 16 * j

            @pl.when(t < NBLK_)
            def _():
                _load_meta(t, pi)
                pltpu.sync_copy(emb_h.at[nidx[pi].at[pl.ds(0, 128)]],
                                nrows[pi].at[pl.ds(0, 128)])
                pltpu.sync_copy(emb_h.at[nidx[pi].at[pl.ds(128, 32)]],
                                nrows[pi].at[pl.ds(128, 32)])

                @pl.loop(0, 16)
                def _(n):
                    a0 = neb[pi][n, pl.ds(0, 16)] + nab[pi][n, pl.ds(0, 16)]
                    for k in range(K_):
                        r = n * K_ + k
                        bc = plsc.load_gather(nwb[pi], [_splat(r)])
                        a0 = a0 + nrows[pi][r, pl.ds(0, 16)] * bc
                    obuf0[n, pl.ds(0, 16)] = a0

                gbase = pl.multiple_of(cbase + t * 16, 16)
                pltpu.sync_copy(obuf0, out_h.at[pl.ds(gbase, 16)])

        @pl.loop(0, NJ_ // 4)
        def _(jo):
            for pi in range(4):
                _block(jo, pi)

    for emb_h, out_h in ((embA_h, outA_h), (embB_h, outB_h)):
        _zero_acc()
        plsc.subcore_barrier()
        _edge_pass(emb_h)
        plsc.subcore_barrier()
        _node_pass(emb_h, out_h)
        plsc.subcore_barrier()


def kernel(user_emb, item_emb, user_sim_neighbor, user_sim_weight,
           item_sim_neighbor, item_sim_weight, graph_edge_index, graph_values):
    emb = jnp.concatenate([user_emb, item_emb], axis=0)
    embA = emb[:, :D2_]
    embB = emb[:, D2_:]
    pad = PAD_E_ - E_
    src = jnp.concatenate([graph_edge_index[1],
                           jnp.zeros((pad,), jnp.int32)]
                          ).reshape(PAD_E_ // EBLK_, EBLK_)
    dst = jnp.concatenate([graph_edge_index[0],
                           jnp.zeros((pad,), jnp.int32)]
                          ).reshape(PAD_E_ // EBLK_, EBLK_)
    val = jnp.concatenate([graph_values, jnp.zeros((pad,), jnp.float32)]
                          ).reshape(PAD_E_ // EBLK_, EBLK_)
    nbr = jnp.concatenate([user_sim_neighbor.reshape(-1),
                           item_sim_neighbor.reshape(-1) + N_USER_])
    wts = jnp.concatenate([user_sim_weight.reshape(-1),
                           item_sim_weight.reshape(-1)])

    mesh = plsc.VectorSubcoreMesh(core_axis_name="c", subcore_axis_name="s",
                                  num_cores=NC_, num_subcores=NS_)
    cp = pltpu.CompilerParams()
    if "needs_layout_passes" in pltpu.CompilerParams.__dataclass_fields__:
        cp = dataclasses.replace(cp, needs_layout_passes=False)
    if "use_tc_tiling_on_sc" in pltpu.CompilerParams.__dataclass_fields__:
        cp = dataclasses.replace(cp, use_tc_tiling_on_sc=False)
    run = pl.kernel(
        _body,
        out_type=(jax.ShapeDtypeStruct((N_, D2_), jnp.float32),
                  jax.ShapeDtypeStruct((N_, D2_), jnp.float32)),
        mesh=mesh,
        scratch_types=[
            pltpu.VMEM_SHARED((ACC_R_, D2_), jnp.float32),
            pltpu.VMEM((SBB_, EBLK_), jnp.int32),
            pltpu.VMEM((SBB_, EBLK_), jnp.int32),
            pltpu.VMEM((SBB_, EBLK_), jnp.int32),
            pltpu.VMEM((SBB_, EBLK_), jnp.int32),
            pltpu.VMEM((SBB_, EBLK_), jnp.float32),
            pltpu.VMEM((SBB_, EBLK_), jnp.float32),
            pltpu.VMEM((EBLK_, D2_), jnp.float32),
            pltpu.VMEM((EBLK_, D2_), jnp.float32),
            pltpu.VMEM((EBLK_, D2_), jnp.float32),
            pltpu.VMEM((EBLK_, D2_), jnp.float32),
            pltpu.VMEM((EBLK_, D2_), jnp.float32),
            pltpu.VMEM((EBLK_, D2_), jnp.float32),
            pltpu.VMEM((EBLK_, D2_), jnp.float32),
            pltpu.VMEM((EBLK_, D2_), jnp.float32),
            pltpu.VMEM((EBLK_,), jnp.int32),
            pltpu.VMEM((EBLK_,), jnp.int32),
            pltpu.VMEM((EBLK_,), jnp.int32),
            pltpu.VMEM((EBLK_,), jnp.int32),
            pltpu.VMEM((16 * K_,), jnp.int32),
            pltpu.VMEM((16 * K_,), jnp.int32),
            pltpu.VMEM((16 * K_,), jnp.int32),
            pltpu.VMEM((16 * K_,), jnp.int32),
            pltpu.VMEM((16 * K_,), jnp.float32),
            pltpu.VMEM((16 * K_,), jnp.float32),
            pltpu.VMEM((16 * K_,), jnp.float32),
            pltpu.VMEM((16 * K_,), jnp.float32),
            pltpu.VMEM((16, D2_), jnp.float32),
            pltpu.VMEM((16, D2_), jnp.float32),
            pltpu.VMEM((16, D2_), jnp.float32),
            pltpu.VMEM((16, D2_), jnp.float32),
            pltpu.VMEM((16, D2_), jnp.float32),
            pltpu.VMEM((16, D2_), jnp.float32),
            pltpu.VMEM((16, D2_), jnp.float32),
            pltpu.VMEM((16, D2_), jnp.float32),
            pltpu.VMEM((16 * K_, D2_), jnp.float32),
            pltpu.VMEM((16 * K_, D2_), jnp.float32),
            pltpu.VMEM((16 * K_, D2_), jnp.float32),
            pltpu.VMEM((16 * K_, D2_), jnp.float32),
            pltpu.VMEM((16, D2_), jnp.float32),
            pltpu.VMEM((16, D2_), jnp.float32),
        ] + [pltpu.SemaphoreType.DMA] * 20,
        compiler_params=cp,
    )
    outA, outB = run(embA, embB, src, dst, val, nbr, wts)
    out = jnp.concatenate([outA, outB], axis=1)
    return (out[:N_USER_], out[N_USER_:])


# speedup vs baseline: 1.0226x; 1.0226x over previous
import dataclasses

import jax
import jax.numpy as jnp
from jax import lax
from jax.experimental import pallas as pl
from jax.experimental.pallas import tpu as pltpu
from jax.experimental.pallas import tpu_sc as plsc

N_USER_ = 50000
N_ITEM_ = 50000
N_ = N_USER_ + N_ITEM_
D_ = 32
D2_ = 16
K_ = 10
E_ = 1600000

NC_ = 2
NS_ = 16
HALF_ = N_ // NC_
ACC_R_ = 51200
TRASH_ = HALF_

EBLK_ = 128
SBB_ = 40
SBE_ = SBB_ * EBLK_
NSB_ = 20
EB_T_ = NSB_ * SBE_
PAD_E_ = NS_ * EB_T_

NBLK_ = HALF_ // 16
NJ_ = 196


def _body(embA_h, embB_h, src_h, dst_h, val_h, nbr_h, wts_h,
          outA_h, outB_h,
          acc,
          msrc0, msrc1, mdst0, mdst1, mval0, mval1,
          rows0, rows1, rows2, rows3,
          osb0, osb1, osb2, osb3,
          lidx0, lidx1, lidx2, lidx3,
          nidx0, nidx1, nidx2, nidx3, nwb0, nwb1, nwb2, nwb3,
          neb0, neb1, neb2, neb3, nab0, nab1, nab2, nab3,
          nr0, nr1, nr2, nr3, obuf0, obuf1,
          gs0, gs1, gs2, gs3, ss0, ss1, ss2, ss3,
          ms0, ms1, ngs0, ngs1, ngs2, ngs3,
          nms0, nms1, nms2, nms3, os0, os1):
    msrc = [msrc0, msrc1]
    mdst = [mdst0, mdst1]
    mval = [mval0, mval1]
    rows = [rows0, rows1, rows2, rows3]
    osb = [osb0, osb1, osb2, osb3]
    lidx = [lidx0, lidx1, lidx2, lidx3]
    nidx = [nidx0, nidx1, nidx2, nidx3]
    nwb = [nwb0, nwb1, nwb2, nwb3]
    neb = [neb0, neb1, neb2, neb3]
    nab = [nab0, nab1, nab2, nab3]
    nrows = [nr0, nr1, nr2, nr3]
    obuf = [obuf0, obuf1]
    gs = [gs0, gs1, gs2, gs3]
    ss = [ss0, ss1, ss2, ss3]
    ms = [ms0, ms1]
    ngs = [ngs0, ngs1, ngs2, ngs3]
    nms = [nms0, nms1, nms2, nms3]
    osm = [os0, os1]

    c = lax.axis_index("c")
    s = lax.axis_index("s")
    cbase = c * HALF_
    tile_e0 = s * EB_T_

    def _splat(i):
        return jnp.full((16,), i, jnp.int32)

    def _zero_acc():
        zero16 = jnp.zeros((16,), jnp.float32)

        @pl.loop(0, EBLK_)
        def _(r):
            osb0[r, pl.ds(0, 16)] = zero16

        @pl.loop(0, ACC_R_ // NS_ // EBLK_)
        def _(jz):
            row0 = pl.multiple_of(s * (ACC_R_ // NS_) + jz * EBLK_, EBLK_)
            pltpu.sync_copy(osb0, acc.at[pl.ds(row0, EBLK_)])

    def _e_issue_meta(j, jj):
        mb = pl.multiple_of((tile_e0 + j * SBE_) // EBLK_, SBB_)
        pltpu.async_copy(src_h.at[pl.ds(mb, SBB_)], msrc[jj], ms[jj])
        pltpu.async_copy(dst_h.at[pl.ds(mb, SBB_)], mdst[jj], ms[jj])
        pltpu.async_copy(val_h.at[pl.ds(mb, SBB_)], mval[jj], ms[jj])

    def _e_wait_meta(jj):
        mb0 = pl.multiple_of(tile_e0 // EBLK_, SBB_)
        pltpu.make_async_copy(src_h.at[pl.ds(mb0, SBB_)], msrc[jj],
                              ms[jj]).wait()
        pltpu.make_async_copy(dst_h.at[pl.ds(mb0, SBB_)], mdst[jj],
                              ms[jj]).wait()
        pltpu.make_async_copy(val_h.at[pl.ds(mb0, SBB_)], mval[jj],
                              ms[jj]).wait()

    def _edge_pass(emb_h):
        def _superblock(jo, jj):
            j = jo * 2 + jj
            _e_wait_meta(jj)

            @pl.loop(0, SBB_)
            def _(b):
                pltpu.sync_copy(emb_h.at[msrc[jj].at[b]], rows0)

                @pl.loop(0, 8)
                def _(g):
                    off = pl.multiple_of(g * 16, 16)
                    d = mdst[jj][b, pl.ds(off, 16)]
                    loc = d - cbase
                    ok = (loc >= 0) & (loc < HALF_)
                    lidx0[pl.ds(off, 16)] = jnp.where(ok, loc, TRASH_)
                    for e in range(16):
                        r = off + e
                        bc = plsc.load_gather(mval[jj],
                                              [_splat(b), _splat(r)])
                        osb0[r, pl.ds(0, 16)] = (
                            rows0[r, pl.ds(0, 16)] * bc)

                pltpu.sync_copy(osb0, acc.at[lidx0], add=True)

            @pl.when(j + 2 < NSB_)
            def _():
                _e_issue_meta(j + 2, jj)

        _e_issue_meta(0, 0)
        _e_issue_meta(1, 1)

        @pl.loop(0, NSB_ // 2)
        def _(jo):
            _superblock(jo, 0)
            _superblock(jo, 1)

    def _node_pass(emb_h, out_h):
        def _load_meta(t, p):
            lbase = pl.multiple_of(t * 16, 16)
            ibase = pl.multiple_of((c * HALF_ + lbase) * K_, 32)
            gbase = pl.multiple_of(cbase + lbase, 16)
            pltpu.sync_copy(nbr_h.at[pl.ds(ibase, 16 * K_)], nidx[p])
            pltpu.sync_copy(wts_h.at[pl.ds(ibase, 16 * K_)], nwb[p])
            pltpu.sync_copy(emb_h.at[pl.ds(gbase, 16)], neb[p])
            pltpu.sync_copy(acc.at[pl.ds(lbase, 16)], nab[p])

        def _block(jo, pi):
            j = jo * 4 + pi
            t = s + 16 * j

            @pl.when(t < NBLK_)
            def _():
                _load_meta(t, pi)
                pltpu.sync_copy(emb_h.at[nidx[pi].at[pl.ds(0, 128)]],
                                nrows[pi].at[pl.ds(0, 128)])
                pltpu.sync_copy(emb_h.at[nidx[pi].at[pl.ds(128, 32)]],
                                nrows[pi].at[pl.ds(128, 32)])

                @pl.loop(0, 16)
                def _(n):
                    a0 = neb[pi][n, pl.ds(0, 16)] + nab[pi][n, pl.ds(0, 16)]
                    for k in range(K_):
                        r = n * K_ + k
                        bc = plsc.load_gather(nwb[pi], [_splat(r)])
                        a0 = a0 + nrows[pi][r, pl.ds(0, 16)] * bc
                    obuf0[n, pl.ds(0, 16)] = a0

                gbase = pl.multiple_of(cbase + t * 16, 16)
                pltpu.sync_copy(obuf0, out_h.at[pl.ds(gbase, 16)])

        @pl.loop(0, NJ_ // 4)
        def _(jo):
            for pi in range(4):
                _block(jo, pi)

    for emb_h, out_h in ((embA_h, outA_h), (embB_h, outB_h)):
        _zero_acc()
        plsc.subcore_barrier()
        _edge_pass(emb_h)
        plsc.subcore_barrier()
        _node_pass(emb_h, out_h)
        plsc.subcore_barrier()


def kernel(user_emb, item_emb, user_sim_neighbor, user_sim_weight,
           item_sim_neighbor, item_sim_weight, graph_edge_index, graph_values):
    emb = jnp.concatenate([user_emb, item_emb], axis=0)
    embA = emb[:, :D2_]
    embB = emb[:, D2_:]
    pad = PAD_E_ - E_
    src = jnp.concatenate([graph_edge_index[1],
                           jnp.zeros((pad,), jnp.int32)]
                          ).reshape(PAD_E_ // EBLK_, EBLK_)
    dst = jnp.concatenate([graph_edge_index[0],
                           jnp.zeros((pad,), jnp.int32)]
                          ).reshape(PAD_E_ // EBLK_, EBLK_)
    val = jnp.concatenate([graph_values, jnp.zeros((pad,), jnp.float32)]
                          ).reshape(PAD_E_ // EBLK_, EBLK_)
    nbr = jnp.concatenate([user_sim_neighbor.reshape(-1),
                           item_sim_neighbor.reshape(-1) + N_USER_])
    wts = jnp.concatenate([user_sim_weight.reshape(-1),
                           item_sim_weight.reshape(-1)])

    mesh = plsc.VectorSubcoreMesh(core_axis_name="c", subcore_axis_name="s",
                                  num_cores=NC_, num_subcores=NS_)
    cp = pltpu.CompilerParams()
    if "needs_layout_passes" in pltpu.CompilerParams.__dataclass_fields__:
        cp = dataclasses.replace(cp, needs_layout_passes=False)
    if "use_tc_tiling_on_sc" in pltpu.CompilerParams.__dataclass_fields__:
        cp = dataclasses.replace(cp, use_tc_tiling_on_sc=False)
    run = pl.kernel(
        _body,
        out_type=(jax.ShapeDtypeStruct((N_, D2_), jnp.float32),
                  jax.ShapeDtypeStruct((N_, D2_), jnp.float32)),
        mesh=mesh,
        scratch_types=[
            pltpu.VMEM_SHARED((ACC_R_, D2_), jnp.float32),
            pltpu.VMEM((SBB_, EBLK_), jnp.int32),
            pltpu.VMEM((SBB_, EBLK_), jnp.int32),
            pltpu.VMEM((SBB_, EBLK_), jnp.int32),
            pltpu.VMEM((SBB_, EBLK_), jnp.int32),
            pltpu.VMEM((SBB_, EBLK_), jnp.float32),
            pltpu.VMEM((SBB_, EBLK_), jnp.float32),
            pltpu.VMEM((EBLK_, D2_), jnp.float32),
            pltpu.VMEM((EBLK_, D2_), jnp.float32),
            pltpu.VMEM((EBLK_, D2_), jnp.float32),
            pltpu.VMEM((EBLK_, D2_), jnp.float32),
            pltpu.VMEM((EBLK_, D2_), jnp.float32),
            pltpu.VMEM((EBLK_, D2_), jnp.float32),
            pltpu.VMEM((EBLK_, D2_), jnp.float32),
            pltpu.VMEM((EBLK_, D2_), jnp.float32),
            pltpu.VMEM((EBLK_,), jnp.int32),
            pltpu.VMEM((EBLK_,), jnp.int32),
            pltpu.VMEM((EBLK_,), jnp.int32),
            pltpu.VMEM((EBLK_,), jnp.int32),
            pltpu.VMEM((16 * K_,), jnp.int32),
            pltpu.VMEM((16 * K_,), jnp.int32),
            pltpu.VMEM((16 * K_,), jnp.int32),
            pltpu.VMEM((16 * K_,), jnp.int32),
            pltpu.VMEM((16 * K_,), jnp.float32),
            pltpu.VMEM((16 * K_,), jnp.float32),
            pltpu.VMEM((16 * K_,), jnp.float32),
            pltpu.VMEM((16 * K_,), jnp.float32),
            pltpu.VMEM((16, D2_), jnp.float32),
            pltpu.VMEM((16, D2_), jnp.float32),
            pltpu.VMEM((16, D2_), jnp.float32),
            pltpu.VMEM((16, D2_), jnp.float32),
            pltpu.VMEM((16, D2_), jnp.float32),
            pltpu.VMEM((16, D2_), jnp.float32),
            pltpu.VMEM((16, D2_), jnp.float32),
            pltpu.VMEM((16, D2_), jnp.float32),
            pltpu.VMEM((16 * K_, D2_), jnp.float32),
            pltpu.VMEM((16 * K_, D2_), jnp.float32),
            pltpu.VMEM((16 * K_, D2_), jnp.float32),
            pltpu.VMEM((16 * K_, D2_), jnp.float32),
            pltpu.VMEM((16, D2_), jnp.float32),
            pltpu.VMEM((16, D2_), jnp.float32),
        ] + [pltpu.SemaphoreType.DMA] * 20,
        compiler_params=cp,
    )
    outA, outB = run(embA, embB, src, dst, val, nbr, wts)
    out = jnp.concatenate([outA, outB], axis=1)
    return (out[:N_USER_], out[N_USER_:])


# speedup vs baseline: 1.2475x; 1.2200x over previous
import dataclasses

import jax
import jax.numpy as jnp
from jax import lax
from jax.experimental import pallas as pl
from jax.experimental.pallas import tpu as pltpu
from jax.experimental.pallas import tpu_sc as plsc

N_USER_ = 50000
N_ITEM_ = 50000
N_ = N_USER_ + N_ITEM_
D_ = 32
D2_ = 16
K_ = 10
E_ = 1600000

NC_ = 2
NS_ = 16
HALF_ = N_ // NC_
ACC_R_ = 51200
TRASH_ = HALF_

EBLK_ = 128
SBB_ = 40
SBE_ = SBB_ * EBLK_
NSB_ = 20
EB_T_ = NSB_ * SBE_
PAD_E_ = NS_ * EB_T_

NBLK_ = HALF_ // 16
NJ_ = 196


def _body(embA_h, embB_h, src_h, dst_h, val_h, nbr_h, wts_h,
          outA_h, outB_h,
          acc,
          msrc0, msrc1, mdst0, mdst1, mval0, mval1,
          rows0, rows1, rows2, rows3,
          osb0, osb1, osb2, osb3,
          lidx0, lidx1, lidx2, lidx3,
          nidx0, nidx1, nidx2, nidx3, nwb0, nwb1, nwb2, nwb3,
          neb0, neb1, neb2, neb3, nab0, nab1, nab2, nab3,
          nr0, nr1, nr2, nr3, obuf0, obuf1,
          gs0, gs1, gs2, gs3, ss0, ss1, ss2, ss3,
          ms0, ms1, ngs0, ngs1, ngs2, ngs3,
          nms0, nms1, nms2, nms3, os0, os1):
    msrc = [msrc0, msrc1]
    mdst = [mdst0, mdst1]
    mval = [mval0, mval1]
    rows = [rows0, rows1, rows2, rows3]
    osb = [osb0, osb1, osb2, osb3]
    lidx = [lidx0, lidx1, lidx2, lidx3]
    nidx = [nidx0, nidx1, nidx2, nidx3]
    nwb = [nwb0, nwb1, nwb2, nwb3]
    neb = [neb0, neb1, neb2, neb3]
    nab = [nab0, nab1, nab2, nab3]
    nrows = [nr0, nr1, nr2, nr3]
    obuf = [obuf0, obuf1]
    gs = [gs0, gs1, gs2, gs3]
    ss = [ss0, ss1, ss2, ss3]
    ms = [ms0, ms1]
    ngs = [ngs0, ngs1, ngs2, ngs3]
    nms = [nms0, nms1, nms2, nms3]
    osm = [os0, os1]

    c = lax.axis_index("c")
    s = lax.axis_index("s")
    cbase = c * HALF_
    tile_e0 = s * EB_T_

    def _splat(i):
        return jnp.full((16,), i, jnp.int32)

    def _zero_acc():
        zero16 = jnp.zeros((16,), jnp.float32)

        @pl.loop(0, EBLK_)
        def _(r):
            osb0[r, pl.ds(0, 16)] = zero16

        @pl.loop(0, ACC_R_ // NS_ // EBLK_)
        def _(jz):
            row0 = pl.multiple_of(s * (ACC_R_ // NS_) + jz * EBLK_, EBLK_)
            pltpu.sync_copy(osb0, acc.at[pl.ds(row0, EBLK_)])

    def _e_issue_meta(j, jj):
        mb = pl.multiple_of((tile_e0 + j * SBE_) // EBLK_, SBB_)
        pltpu.async_copy(src_h.at[pl.ds(mb, SBB_)], msrc[jj], ms[jj])
        pltpu.async_copy(dst_h.at[pl.ds(mb, SBB_)], mdst[jj], ms[jj])
        pltpu.async_copy(val_h.at[pl.ds(mb, SBB_)], mval[jj], ms[jj])

    def _e_wait_meta(jj):
        mb0 = pl.multiple_of(tile_e0 // EBLK_, SBB_)
        pltpu.make_async_copy(src_h.at[pl.ds(mb0, SBB_)], msrc[jj],
                              ms[jj]).wait()
        pltpu.make_async_copy(dst_h.at[pl.ds(mb0, SBB_)], mdst[jj],
                              ms[jj]).wait()
        pltpu.make_async_copy(val_h.at[pl.ds(mb0, SBB_)], mval[jj],
                              ms[jj]).wait()

    def _edge_pass(emb_h):
        def _gath(jj, b, slot):
            pltpu.async_copy(emb_h.at[msrc[jj].at[b]], rows[slot], gs[slot])

        def _gath_wait(slot):
            pltpu.make_async_copy(emb_h.at[msrc[0].at[0]], rows[slot],
                                  gs[slot]).wait()

        def _block(jj, b, slot):
            @pl.loop(0, 8)
            def _(g):
                off = pl.multiple_of(g * 16, 16)
                d = mdst[jj][b, pl.ds(off, 16)]
                loc = d - cbase
                ok = (loc >= 0) & (loc < HALF_)
                lidx[slot][pl.ds(off, 16)] = jnp.where(ok, loc, TRASH_)
                for e in range(16):
                    r = off + e
                    bc = plsc.load_gather(mval[jj],
                                          [_splat(b), _splat(r)])
                    osb[slot][r, pl.ds(0, 16)] = (
                        rows[slot][r, pl.ds(0, 16)] * bc)

        def _superblock(jo, jj):
            j = jo * 2 + jj
            _e_wait_meta(jj)

            @pl.loop(0, (SBB_ + 4) // 4)
            def _(eq):
                for b4 in range(4):
                    v = eq * 4 + b4

                    @pl.when((v >= 4) & (v < SBB_ + 4))
                    def _():
                        _gath_wait(b4)
                        _block(jj, v - 4, b4)
                        pltpu.sync_copy(osb[b4], acc.at[lidx[b4]],
                                        add=True)

                    @pl.when(v < SBB_)
                    def _():
                        _gath(jj, v, b4)

            @pl.when(j + 2 < NSB_)
            def _():
                _e_issue_meta(j + 2, jj)

        _e_issue_meta(0, 0)
        _e_issue_meta(1, 1)

        @pl.loop(0, NSB_ // 2)
        def _(jo):
            _superblock(jo, 0)
            _superblock(jo, 1)

    def _node_pass(emb_h, out_h):
        def _load_meta(t, p):
            lbase = pl.multiple_of(t * 16, 16)
            ibase = pl.multiple_of((c * HALF_ + lbase) * K_, 32)
            gbase = pl.multiple_of(cbase + lbase, 16)
            pltpu.sync_copy(nbr_h.at[pl.ds(ibase, 16 * K_)], nidx[p])
            pltpu.sync_copy(wts_h.at[pl.ds(ibase, 16 * K_)], nwb[p])
            pltpu.sync_copy(emb_h.at[pl.ds(gbase, 16)], neb[p])
            pltpu.sync_copy(acc.at[pl.ds(lbase, 16)], nab[p])

        def _block(jo, pi):
            j = jo * 4 + pi
            t = s + 16 * j

            @pl.when(t < NBLK_)
            def _():
                _load_meta(t, pi)
                pltpu.sync_copy(emb_h.at[nidx[pi].at[pl.ds(0, 128)]],
                                nrows[pi].at[pl.ds(0, 128)])
                pltpu.sync_copy(emb_h.at[nidx[pi].at[pl.ds(128, 32)]],
                                nrows[pi].at[pl.ds(128, 32)])

                @pl.loop(0, 16)
                def _(n):
                    a0 = neb[pi][n, pl.ds(0, 16)] + nab[pi][n, pl.ds(0, 16)]
                    for k in range(K_):
                        r = n * K_ + k
                        bc = plsc.load_gather(nwb[pi], [_splat(r)])
                        a0 = a0 + nrows[pi][r, pl.ds(0, 16)] * bc
                    obuf0[n, pl.ds(0, 16)] = a0

                gbase = pl.multiple_of(cbase + t * 16, 16)
                pltpu.sync_copy(obuf0, out_h.at[pl.ds(gbase, 16)])

        @pl.loop(0, NJ_ // 4)
        def _(jo):
            for pi in range(4):
                _block(jo, pi)

    for emb_h, out_h in ((embA_h, outA_h), (embB_h, outB_h)):
        _zero_acc()
        plsc.subcore_barrier()
        _edge_pass(emb_h)
        plsc.subcore_barrier()
        _node_pass(emb_h, out_h)
        plsc.subcore_barrier()


def kernel(user_emb, item_emb, user_sim_neighbor, user_sim_weight,
           item_sim_neighbor, item_sim_weight, graph_edge_index, graph_values):
    emb = jnp.concatenate([user_emb, item_emb], axis=0)
    embA = emb[:, :D2_]
    embB = emb[:, D2_:]
    pad = PAD_E_ - E_
    src = jnp.concatenate([graph_edge_index[1],
                           jnp.zeros((pad,), jnp.int32)]
                          ).reshape(PAD_E_ // EBLK_, EBLK_)
    dst = jnp.concatenate([graph_edge_index[0],
                           jnp.zeros((pad,), jnp.int32)]
                          ).reshape(PAD_E_ // EBLK_, EBLK_)
    val = jnp.concatenate([graph_values, jnp.zeros((pad,), jnp.float32)]
                          ).reshape(PAD_E_ // EBLK_, EBLK_)
    nbr = jnp.concatenate([user_sim_neighbor.reshape(-1),
                           item_sim_neighbor.reshape(-1) + N_USER_])
    wts = jnp.concatenate([user_sim_weight.reshape(-1),
                           item_sim_weight.reshape(-1)])

    mesh = plsc.VectorSubcoreMesh(core_axis_name="c", subcore_axis_name="s",
                                  num_cores=NC_, num_subcores=NS_)
    cp = pltpu.CompilerParams()
    if "needs_layout_passes" in pltpu.CompilerParams.__dataclass_fields__:
        cp = dataclasses.replace(cp, needs_layout_passes=False)
    if "use_tc_tiling_on_sc" in pltpu.CompilerParams.__dataclass_fields__:
        cp = dataclasses.replace(cp, use_tc_tiling_on_sc=False)
    run = pl.kernel(
        _body,
        out_type=(jax.ShapeDtypeStruct((N_, D2_), jnp.float32),
                  jax.ShapeDtypeStruct((N_, D2_), jnp.float32)),
        mesh=mesh,
        scratch_types=[
            pltpu.VMEM_SHARED((ACC_R_, D2_), jnp.float32),
            pltpu.VMEM((SBB_, EBLK_), jnp.int32),
            pltpu.VMEM((SBB_, EBLK_), jnp.int32),
            pltpu.VMEM((SBB_, EBLK_), jnp.int32),
            pltpu.VMEM((SBB_, EBLK_), jnp.int32),
            pltpu.VMEM((SBB_, EBLK_), jnp.float32),
            pltpu.VMEM((SBB_, EBLK_), jnp.float32),
            pltpu.VMEM((EBLK_, D2_), jnp.float32),
            pltpu.VMEM((EBLK_, D2_), jnp.float32),
            pltpu.VMEM((EBLK_, D2_), jnp.float32),
            pltpu.VMEM((EBLK_, D2_), jnp.float32),
            pltpu.VMEM((EBLK_, D2_), jnp.float32),
            pltpu.VMEM((EBLK_, D2_), jnp.float32),
            pltpu.VMEM((EBLK_, D2_), jnp.float32),
            pltpu.VMEM((EBLK_, D2_), jnp.float32),
            pltpu.VMEM((EBLK_,), jnp.int32),
            pltpu.VMEM((EBLK_,), jnp.int32),
            pltpu.VMEM((EBLK_,), jnp.int32),
            pltpu.VMEM((EBLK_,), jnp.int32),
            pltpu.VMEM((16 * K_,), jnp.int32),
            pltpu.VMEM((16 * K_,), jnp.int32),
            pltpu.VMEM((16 * K_,), jnp.int32),
            pltpu.VMEM((16 * K_,), jnp.int32),
            pltpu.VMEM((16 * K_,), jnp.float32),
            pltpu.VMEM((16 * K_,), jnp.float32),
            pltpu.VMEM((16 * K_,), jnp.float32),
            pltpu.VMEM((16 * K_,), jnp.float32),
            pltpu.VMEM((16, D2_), jnp.float32),
            pltpu.VMEM((16, D2_), jnp.float32),
            pltpu.VMEM((16, D2_), jnp.float32),
            pltpu.VMEM((16, D2_), jnp.float32),
            pltpu.VMEM((16, D2_), jnp.float32),
            pltpu.VMEM((16, D2_), jnp.float32),
            pltpu.VMEM((16, D2_), jnp.float32),
            pltpu.VMEM((16, D2_), jnp.float32),
            pltpu.VMEM((16 * K_, D2_), jnp.float32),
            pltpu.VMEM((16 * K_, D2_), jnp.float32),
            pltpu.VMEM((16 * K_, D2_), jnp.float32),
            pltpu.VMEM((16 * K_, D2_), jnp.float32),
            pltpu.VMEM((16, D2_), jnp.float32),
            pltpu.VMEM((16, D2_), jnp.float32),
        ] + [pltpu.SemaphoreType.DMA] * 20,
        compiler_params=cp,
    )
    outA, outB = run(embA, embB, src, dst, val, nbr, wts)
    out = jnp.concatenate([outA, outB], axis=1)
    return (out[:N_USER_], out[N_USER_:])
